# baseline (speedup 1.0000x reference)
import functools

import jax
import jax.numpy as jnp
from jax.experimental import pallas as pl
from jax.experimental.pallas import tpu as pltpu

_E = 64
_K = 2
_CAP_F = 1.25
_ZC = 0.001
_AC = 0.01
_HB = 512
_B = 2 * _HB


def _router_kernel(hsa_ref, hsb_ref, gw_ref, w_ref, id_ref, keep_ref, aux_ref,
                   carry_ref, psum_ref, cnt_ref, z_ref,
                   *, nsteps, T, capacity):
    i = pl.program_id(0)

    @pl.when(i == 0)
    def _init():
        carry_ref[...] = jnp.zeros_like(carry_ref)
        psum_ref[...] = jnp.zeros_like(psum_ref)
        cnt_ref[...] = jnp.zeros_like(cnt_ref)
        z_ref[...] = jnp.zeros_like(z_ref)

    la = jax.lax.dot_general(
        hsa_ref[...], gw_ref[...],
        dimension_numbers=(((1,), (1,)), ((), ())),
        preferred_element_type=jnp.float32)
    lb = jax.lax.dot_general(
        hsb_ref[...], gw_ref[...],
        dimension_numbers=(((1,), (1,)), ((), ())),
        preferred_element_type=jnp.float32)
    logits = jnp.concatenate([la, lb], axis=0)

    m1 = jnp.max(logits, axis=1, keepdims=True)
    ex = jnp.exp(logits - m1)
    sumex = jnp.sum(ex, axis=1, keepdims=True)
    probs = ex / sumex
    lse = m1 + jnp.log(sumex)

    col = jax.lax.broadcasted_iota(jnp.int32, logits.shape, 1)
    top1 = jnp.min(jnp.where(logits == m1, col, _E), axis=1, keepdims=True)
    oh0 = (col == top1).astype(jnp.float32)
    masked = jnp.where(col == top1, -jnp.inf, logits)
    m2 = jnp.max(masked, axis=1, keepdims=True)
    top2 = jnp.min(jnp.where(masked == m2, col, _E), axis=1, keepdims=True)
    oh1 = (col == top2).astype(jnp.float32)

    v1 = jnp.sum(probs * oh0, axis=1, keepdims=True)
    v2 = jnp.sum(probs * oh1, axis=1, keepdims=True)
    denom = v1 + v2

    cnt = oh0 + oh1
    r = jax.lax.broadcasted_iota(jnp.int32, (_B, _B), 0)
    c = jax.lax.broadcasted_iota(jnp.int32, (_B, _B), 1)
    ltri = (c < r).astype(jnp.float32)
    prefix = jnp.dot(ltri, cnt, preferred_element_type=jnp.float32)
    prefix = prefix + carry_ref[...]
    pos0 = jnp.sum(prefix * oh0, axis=1, keepdims=True)
    pos1 = jnp.sum(prefix * oh1, axis=1, keepdims=True)
    keep0 = (pos0 < capacity).astype(jnp.float32)
    keep1 = (pos1 < capacity).astype(jnp.float32)

    w_ref[...] = jnp.concatenate([v1 / denom, v2 / denom], axis=1)
    id_ref[...] = jnp.concatenate([top1, top2], axis=1)
    keep_ref[...] = jnp.concatenate([keep0, keep1], axis=1)

    carry_ref[...] = carry_ref[...] + jnp.sum(cnt, axis=0, keepdims=True)
    psum_ref[...] = psum_ref[...] + jnp.sum(probs, axis=0, keepdims=True)
    cnt_ref[...] = cnt_ref[...] + jnp.sum(oh0, axis=0, keepdims=True)
    z_ref[...] = z_ref[...] + jnp.sum(lse * lse).reshape(1, 1)

    @pl.when(i == nsteps - 1)
    def _finish():
        z_loss = z_ref[...] / T
        mean_probs = psum_ref[...] / T
        expert_frac = cnt_ref[...] / T
        aux_val = _E * jnp.sum(expert_frac * mean_probs).reshape(1, 1)
        aux_ref[...] = _ZC * z_loss + _AC * aux_val


def kernel(hidden_states, gate_weight):
    T, D = hidden_states.shape
    capacity = float(int(_CAP_F * T * _K / _E))
    nsteps = T // _B

    out_shape = [
        jax.ShapeDtypeStruct((T, _K), jnp.float32),
        jax.ShapeDtypeStruct((T, _K), jnp.int32),
        jax.ShapeDtypeStruct((T, _K), jnp.float32),
        jax.ShapeDtypeStruct((1, 1), jnp.float32),
    ]
    w, ids, keep, aux = pl.pallas_call(
        functools.partial(_router_kernel, nsteps=nsteps, T=float(T),
                          capacity=capacity),
        grid=(nsteps,),
        in_specs=[
            pl.BlockSpec((_HB, D), lambda i: (2 * i, 0)),
            pl.BlockSpec((_HB, D), lambda i: (2 * i + 1, 0)),
            pl.BlockSpec((_E, D), lambda i: (0, 0)),
        ],
        out_specs=[
            pl.BlockSpec((_B, _K), lambda i: (i, 0)),
            pl.BlockSpec((_B, _K), lambda i: (i, 0)),
            pl.BlockSpec((_B, _K), lambda i: (i, 0)),
            pl.BlockSpec((1, 1), lambda i: (0, 0)),
        ],
        out_shape=out_shape,
        scratch_shapes=[
            pltpu.VMEM((1, _E), jnp.float32),
            pltpu.VMEM((1, _E), jnp.float32),
            pltpu.VMEM((1, _E), jnp.float32),
            pltpu.VMEM((1, 1), jnp.float32),
        ],
    )(hidden_states, hidden_states, gate_weight)
    return w, ids, keep > 0.5, aux[0, 0]


# speedup vs baseline: 1.0066x; 1.0066x over previous
import functools

import jax
import jax.numpy as jnp
from jax.experimental import pallas as pl
from jax.experimental.pallas import tpu as pltpu

_E = 64
_K = 2
_CAP_F = 1.25
_ZC = 0.001
_AC = 0.01
_HB = 256
_B = 4 * _HB


def _router_kernel(hsa_ref, hsb_ref, hsc_ref, hsd_ref, gw_ref, w_ref, id_ref, keep_ref, aux_ref,
                   carry_ref, psum_ref, cnt_ref, z_ref,
                   *, nsteps, T, capacity):
    i = pl.program_id(0)

    @pl.when(i == 0)
    def _init():
        carry_ref[...] = jnp.zeros_like(carry_ref)
        psum_ref[...] = jnp.zeros_like(psum_ref)
        cnt_ref[...] = jnp.zeros_like(cnt_ref)
        z_ref[...] = jnp.zeros_like(z_ref)

    la = jax.lax.dot_general(
        hsa_ref[...], gw_ref[...],
        dimension_numbers=(((1,), (1,)), ((), ())),
        preferred_element_type=jnp.float32)
    lb = jax.lax.dot_general(
        hsb_ref[...], gw_ref[...],
        dimension_numbers=(((1,), (1,)), ((), ())),
        preferred_element_type=jnp.float32)
    lc = jax.lax.dot_general(
        hsc_ref[...], gw_ref[...],
        dimension_numbers=(((1,), (1,)), ((), ())),
        preferred_element_type=jnp.float32)
    ld = jax.lax.dot_general(
        hsd_ref[...], gw_ref[...],
        dimension_numbers=(((1,), (1,)), ((), ())),
        preferred_element_type=jnp.float32)
    logits = jnp.concatenate([la, lb, lc, ld], axis=0)

    m1 = jnp.max(logits, axis=1, keepdims=True)
    ex = jnp.exp(logits - m1)
    sumex = jnp.sum(ex, axis=1, keepdims=True)
    probs = ex / sumex
    lse = m1 + jnp.log(sumex)

    col = jax.lax.broadcasted_iota(jnp.int32, logits.shape, 1)
    top1 = jnp.min(jnp.where(logits == m1, col, _E), axis=1, keepdims=True)
    oh0 = (col == top1).astype(jnp.float32)
    masked = jnp.where(col == top1, -jnp.inf, logits)
    m2 = jnp.max(masked, axis=1, keepdims=True)
    top2 = jnp.min(jnp.where(masked == m2, col, _E), axis=1, keepdims=True)
    oh1 = (col == top2).astype(jnp.float32)

    v1 = jnp.sum(probs * oh0, axis=1, keepdims=True)
    v2 = jnp.sum(probs * oh1, axis=1, keepdims=True)
    denom = v1 + v2

    cnt = oh0 + oh1
    r = jax.lax.broadcasted_iota(jnp.int32, (_B, _B), 0)
    c = jax.lax.broadcasted_iota(jnp.int32, (_B, _B), 1)
    ltri = (c < r).astype(jnp.float32)
    prefix = jnp.dot(ltri, cnt, preferred_element_type=jnp.float32)
    prefix = prefix + carry_ref[...]
    pos0 = jnp.sum(prefix * oh0, axis=1, keepdims=True)
    pos1 = jnp.sum(prefix * oh1, axis=1, keepdims=True)
    keep0 = (pos0 < capacity).astype(jnp.float32)
    keep1 = (pos1 < capacity).astype(jnp.float32)

    w_ref[...] = jnp.concatenate([v1 / denom, v2 / denom], axis=1)
    id_ref[...] = jnp.concatenate([top1, top2], axis=1)
    keep_ref[...] = jnp.concatenate([keep0, keep1], axis=1)

    carry_ref[...] = carry_ref[...] + jnp.sum(cnt, axis=0, keepdims=True)
    psum_ref[...] = psum_ref[...] + jnp.sum(probs, axis=0, keepdims=True)
    cnt_ref[...] = cnt_ref[...] + jnp.sum(oh0, axis=0, keepdims=True)
    z_ref[...] = z_ref[...] + jnp.sum(lse * lse).reshape(1, 1)

    @pl.when(i == nsteps - 1)
    def _finish():
        z_loss = z_ref[...] / T
        mean_probs = psum_ref[...] / T
        expert_frac = cnt_ref[...] / T
        aux_val = _E * jnp.sum(expert_frac * mean_probs).reshape(1, 1)
        aux_ref[...] = _ZC * z_loss + _AC * aux_val


def kernel(hidden_states, gate_weight):
    T, D = hidden_states.shape
    capacity = float(int(_CAP_F * T * _K / _E))
    nsteps = T // _B

    out_shape = [
        jax.ShapeDtypeStruct((T, _K), jnp.float32),
        jax.ShapeDtypeStruct((T, _K), jnp.int32),
        jax.ShapeDtypeStruct((T, _K), jnp.float32),
        jax.ShapeDtypeStruct((1, 1), jnp.float32),
    ]
    w, ids, keep, aux = pl.pallas_call(
        functools.partial(_router_kernel, nsteps=nsteps, T=float(T),
                          capacity=capacity),
        grid=(nsteps,),
        in_specs=[
            pl.BlockSpec((_HB, D), lambda i: (4 * i, 0)),
            pl.BlockSpec((_HB, D), lambda i: (4 * i + 1, 0)),
            pl.BlockSpec((_HB, D), lambda i: (4 * i + 2, 0)),
            pl.BlockSpec((_HB, D), lambda i: (4 * i + 3, 0)),
            pl.BlockSpec((_E, D), lambda i: (0, 0)),
        ],
        out_specs=[
            pl.BlockSpec((_B, _K), lambda i: (i, 0)),
            pl.BlockSpec((_B, _K), lambda i: (i, 0)),
            pl.BlockSpec((_B, _K), lambda i: (i, 0)),
            pl.BlockSpec((1, 1), lambda i: (0, 0)),
        ],
        out_shape=out_shape,
        scratch_shapes=[
            pltpu.VMEM((1, _E), jnp.float32),
            pltpu.VMEM((1, _E), jnp.float32),
            pltpu.VMEM((1, _E), jnp.float32),
            pltpu.VMEM((1, 1), jnp.float32),
        ],
    )(hidden_states, hidden_states, hidden_states, hidden_states, gate_weight)
    return w, ids, keep > 0.5, aux[0, 0]


# speedup vs baseline: 1.0134x; 1.0067x over previous
import functools

import jax
import jax.numpy as jnp
from jax.experimental import pallas as pl
from jax.experimental.pallas import tpu as pltpu

_E = 64
_K = 2
_CAP_F = 1.25
_ZC = 0.001
_AC = 0.01
_HB = 256
_B = 4 * _HB


def _router_kernel(hsa_ref, hsb_ref, hsc_ref, hsd_ref, gw_ref, w_ref, id_ref, keep_ref, aux_ref,
                   carry_ref, psum_ref, cnt_ref, z_ref,
                   *, nsteps, T, capacity):
    i = pl.program_id(0)

    @pl.when(i == 0)
    def _init():
        carry_ref[...] = jnp.zeros_like(carry_ref)
        psum_ref[...] = jnp.zeros_like(psum_ref)
        cnt_ref[...] = jnp.zeros_like(cnt_ref)
        z_ref[...] = jnp.zeros_like(z_ref)

    la = jax.lax.dot_general(
        hsa_ref[...], gw_ref[...],
        dimension_numbers=(((1,), (1,)), ((), ())),
        preferred_element_type=jnp.float32)
    lb = jax.lax.dot_general(
        hsb_ref[...], gw_ref[...],
        dimension_numbers=(((1,), (1,)), ((), ())),
        preferred_element_type=jnp.float32)
    lc = jax.lax.dot_general(
        hsc_ref[...], gw_ref[...],
        dimension_numbers=(((1,), (1,)), ((), ())),
        preferred_element_type=jnp.float32)
    ld = jax.lax.dot_general(
        hsd_ref[...], gw_ref[...],
        dimension_numbers=(((1,), (1,)), ((), ())),
        preferred_element_type=jnp.float32)
    logits = jnp.concatenate([la, lb, lc, ld], axis=0)

    m1 = jnp.max(logits, axis=1, keepdims=True)
    ex = jnp.exp(logits - m1)
    sumex = jnp.sum(ex, axis=1, keepdims=True)
    probs = ex / sumex
    lse = m1 + jnp.log(sumex)

    col = jax.lax.broadcasted_iota(jnp.int32, logits.shape, 1)
    top1 = jnp.min(jnp.where(logits == m1, col, _E), axis=1, keepdims=True)
    oh0 = (col == top1).astype(jnp.float32)
    masked = jnp.where(col == top1, -jnp.inf, logits)
    m2 = jnp.max(masked, axis=1, keepdims=True)
    top2 = jnp.min(jnp.where(masked == m2, col, _E), axis=1, keepdims=True)
    oh1 = (col == top2).astype(jnp.float32)

    v1 = jnp.sum(probs * oh0, axis=1, keepdims=True)
    v2 = jnp.sum(probs * oh1, axis=1, keepdims=True)
    denom = v1 + v2

    cnt = oh0 + oh1
    h = _B // 2
    r = jax.lax.broadcasted_iota(jnp.int32, (h, h), 0)
    c = jax.lax.broadcasted_iota(jnp.int32, (h, h), 1)
    ltri = (c < r).astype(jnp.bfloat16)
    cnt_b = cnt.astype(jnp.bfloat16)
    p1 = jnp.dot(ltri, cnt_b[:h], preferred_element_type=jnp.float32)
    s1 = jnp.sum(cnt[:h], axis=0, keepdims=True)
    p2 = jnp.dot(ltri, cnt_b[h:], preferred_element_type=jnp.float32) + s1
    prefix = jnp.concatenate([p1, p2], axis=0)
    prefix = prefix + carry_ref[...]
    pos0 = jnp.sum(prefix * oh0, axis=1, keepdims=True)
    pos1 = jnp.sum(prefix * oh1, axis=1, keepdims=True)
    keep0 = (pos0 < capacity).astype(jnp.float32)
    keep1 = (pos1 < capacity).astype(jnp.float32)

    w_ref[...] = jnp.concatenate([v1 / denom, v2 / denom], axis=1)
    id_ref[...] = jnp.concatenate([top1, top2], axis=1)
    keep_ref[...] = jnp.concatenate([keep0, keep1], axis=1)

    carry_ref[...] = carry_ref[...] + jnp.sum(cnt, axis=0, keepdims=True)
    psum_ref[...] = psum_ref[...] + jnp.sum(probs, axis=0, keepdims=True)
    cnt_ref[...] = cnt_ref[...] + jnp.sum(oh0, axis=0, keepdims=True)
    z_ref[...] = z_ref[...] + jnp.sum(lse * lse).reshape(1, 1)

    @pl.when(i == nsteps - 1)
    def _finish():
        z_loss = z_ref[...] / T
        mean_probs = psum_ref[...] / T
        expert_frac = cnt_ref[...] / T
        aux_val = _E * jnp.sum(expert_frac * mean_probs).reshape(1, 1)
        aux_ref[...] = _ZC * z_loss + _AC * aux_val


def kernel(hidden_states, gate_weight):
    T, D = hidden_states.shape
    capacity = float(int(_CAP_F * T * _K / _E))
    nsteps = T // _B

    out_shape = [
        jax.ShapeDtypeStruct((T, _K), jnp.float32),
        jax.ShapeDtypeStruct((T, _K), jnp.int32),
        jax.ShapeDtypeStruct((T, _K), jnp.float32),
        jax.ShapeDtypeStruct((1, 1), jnp.float32),
    ]
    w, ids, keep, aux = pl.pallas_call(
        functools.partial(_router_kernel, nsteps=nsteps, T=float(T),
                          capacity=capacity),
        grid=(nsteps,),
        in_specs=[
            pl.BlockSpec((_HB, D), lambda i: (4 * i, 0)),
            pl.BlockSpec((_HB, D), lambda i: (4 * i + 1, 0)),
            pl.BlockSpec((_HB, D), lambda i: (4 * i + 2, 0)),
            pl.BlockSpec((_HB, D), lambda i: (4 * i + 3, 0)),
            pl.BlockSpec((_E, D), lambda i: (0, 0)),
        ],
        out_specs=[
            pl.BlockSpec((_B, _K), lambda i: (i, 0)),
            pl.BlockSpec((_B, _K), lambda i: (i, 0)),
            pl.BlockSpec((_B, _K), lambda i: (i, 0)),
            pl.BlockSpec((1, 1), lambda i: (0, 0)),
        ],
        out_shape=out_shape,
        scratch_shapes=[
            pltpu.VMEM((1, _E), jnp.float32),
            pltpu.VMEM((1, _E), jnp.float32),
            pltpu.VMEM((1, _E), jnp.float32),
            pltpu.VMEM((1, 1), jnp.float32),
        ],
    )(hidden_states, hidden_states, hidden_states, hidden_states, gate_weight)
    return w, ids, keep > 0.5, aux[0, 0]


# speedup vs baseline: 1.0141x; 1.0007x over previous
import functools

import jax
import jax.numpy as jnp
from jax.experimental import pallas as pl
from jax.experimental.pallas import tpu as pltpu

_E = 64
_K = 2
_CAP_F = 1.25
_ZC = 0.001
_AC = 0.01
_HB = 256
_B = 4 * _HB


def _router_kernel(hsa_ref, hsb_ref, hsc_ref, hsd_ref, gw_ref, w_ref, id_ref, keep_ref, aux_ref,
                   carry_ref, psum_ref, cnt_ref, z_ref,
                   *, nsteps, T, capacity):
    i = pl.program_id(0)

    @pl.when(i == 0)
    def _init():
        carry_ref[...] = jnp.zeros_like(carry_ref)
        psum_ref[...] = jnp.zeros_like(psum_ref)
        cnt_ref[...] = jnp.zeros_like(cnt_ref)
        z_ref[...] = jnp.zeros_like(z_ref)

    la = jax.lax.dot_general(
        hsa_ref[...], gw_ref[...],
        dimension_numbers=(((1,), (1,)), ((), ())),
        preferred_element_type=jnp.float32)
    lb = jax.lax.dot_general(
        hsb_ref[...], gw_ref[...],
        dimension_numbers=(((1,), (1,)), ((), ())),
        preferred_element_type=jnp.float32)
    lc = jax.lax.dot_general(
        hsc_ref[...], gw_ref[...],
        dimension_numbers=(((1,), (1,)), ((), ())),
        preferred_element_type=jnp.float32)
    ld = jax.lax.dot_general(
        hsd_ref[...], gw_ref[...],
        dimension_numbers=(((1,), (1,)), ((), ())),
        preferred_element_type=jnp.float32)
    logits = jnp.concatenate([la, lb, lc, ld], axis=0)

    m1 = jnp.max(logits, axis=1, keepdims=True)
    ex = jnp.exp(logits - m1)
    sumex = jnp.sum(ex, axis=1, keepdims=True)
    probs = ex / sumex
    lse = m1 + jnp.log(sumex)

    col = jax.lax.broadcasted_iota(jnp.int32, logits.shape, 1)
    top1 = jnp.min(jnp.where(logits == m1, col, _E), axis=1, keepdims=True)
    oh0 = (col == top1).astype(jnp.float32)
    masked = jnp.where(col == top1, -jnp.inf, logits)
    m2 = jnp.max(masked, axis=1, keepdims=True)
    top2 = jnp.min(jnp.where(masked == m2, col, _E), axis=1, keepdims=True)
    oh1 = (col == top2).astype(jnp.float32)

    v1 = jnp.sum(probs * oh0, axis=1, keepdims=True)
    v2 = jnp.sum(probs * oh1, axis=1, keepdims=True)
    denom = v1 + v2

    cnt = oh0 + oh1
    h = _B // 2
    r = jax.lax.broadcasted_iota(jnp.int32, (h, h), 0)
    c = jax.lax.broadcasted_iota(jnp.int32, (h, h), 1)
    ltri = (c < r).astype(jnp.bfloat16)
    cnt_b = cnt.astype(jnp.bfloat16)
    ones_h = jnp.ones((1, h), jnp.bfloat16)
    p1 = jnp.dot(ltri, cnt_b[:h], preferred_element_type=jnp.float32)
    s1 = jnp.dot(ones_h, cnt_b[:h], preferred_element_type=jnp.float32)
    s2 = jnp.dot(ones_h, cnt_b[h:], preferred_element_type=jnp.float32)
    p2 = jnp.dot(ltri, cnt_b[h:], preferred_element_type=jnp.float32) + s1
    prefix = jnp.concatenate([p1, p2], axis=0)
    prefix = prefix + carry_ref[...]
    pos0 = jnp.sum(prefix * oh0, axis=1, keepdims=True)
    pos1 = jnp.sum(prefix * oh1, axis=1, keepdims=True)
    keep0 = (pos0 < capacity).astype(jnp.float32)
    keep1 = (pos1 < capacity).astype(jnp.float32)

    w_ref[...] = jnp.concatenate([v1 / denom, v2 / denom], axis=1)
    id_ref[...] = jnp.concatenate([top1, top2], axis=1)
    keep_ref[...] = jnp.concatenate([keep0, keep1], axis=1)

    ones_b = jnp.ones((1, _B), jnp.float32)
    carry_ref[...] = carry_ref[...] + s1 + s2
    psum_ref[...] = psum_ref[...] + jnp.dot(
        ones_b, probs, preferred_element_type=jnp.float32)
    cnt_ref[...] = cnt_ref[...] + jnp.dot(
        ones_b, oh0, preferred_element_type=jnp.float32)
    z_ref[...] = z_ref[...] + jnp.sum(lse * lse).reshape(1, 1)

    @pl.when(i == nsteps - 1)
    def _finish():
        z_loss = z_ref[...] / T
        mean_probs = psum_ref[...] / T
        expert_frac = cnt_ref[...] / T
        aux_val = _E * jnp.sum(expert_frac * mean_probs).reshape(1, 1)
        aux_ref[...] = _ZC * z_loss + _AC * aux_val


def kernel(hidden_states, gate_weight):
    T, D = hidden_states.shape
    capacity = float(int(_CAP_F * T * _K / _E))
    nsteps = T // _B

    out_shape = [
        jax.ShapeDtypeStruct((T, _K), jnp.float32),
        jax.ShapeDtypeStruct((T, _K), jnp.int32),
        jax.ShapeDtypeStruct((T, _K), jnp.float32),
        jax.ShapeDtypeStruct((1, 1), jnp.float32),
    ]
    w, ids, keep, aux = pl.pallas_call(
        functools.partial(_router_kernel, nsteps=nsteps, T=float(T),
                          capacity=capacity),
        grid=(nsteps,),
        in_specs=[
            pl.BlockSpec((_HB, D), lambda i: (4 * i, 0)),
            pl.BlockSpec((_HB, D), lambda i: (4 * i + 1, 0)),
            pl.BlockSpec((_HB, D), lambda i: (4 * i + 2, 0)),
            pl.BlockSpec((_HB, D), lambda i: (4 * i + 3, 0)),
            pl.BlockSpec((_E, D), lambda i: (0, 0)),
        ],
        out_specs=[
            pl.BlockSpec((_B, _K), lambda i: (i, 0)),
            pl.BlockSpec((_B, _K), lambda i: (i, 0)),
            pl.BlockSpec((_B, _K), lambda i: (i, 0)),
            pl.BlockSpec((1, 1), lambda i: (0, 0)),
        ],
        out_shape=out_shape,
        scratch_shapes=[
            pltpu.VMEM((1, _E), jnp.float32),
            pltpu.VMEM((1, _E), jnp.float32),
            pltpu.VMEM((1, _E), jnp.float32),
            pltpu.VMEM((1, 1), jnp.float32),
        ],
    )(hidden_states, hidden_states, hidden_states, hidden_states, gate_weight)
    return w, ids, keep > 0.5, aux[0, 0]
